# baseline (speedup 1.0000x reference)
import functools

import jax
import jax.numpy as jnp
from jax import lax
from jax.experimental import pallas as pl
from jax.experimental.pallas import tpu as pltpu
from jax.experimental.pallas import tpu_sc as plsc

N = 10000
E = 320000
D_IN = 128
HID = 8
HEADS = 8
D_OUT = 64

NC = 2
NS = 16
NT = NC * NS
CK = 128
CPT = 80
EPAD = NT * CPT * CK
NPAD = 10112
RPT = NPAD // NS
ACCW = 72
BN = 1000
GRID = N // BN


def _leaky_exp(z):
    return jnp.exp(jnp.maximum(z, 0.2 * z))


def _tc_pre_body(x_ref, w_ref, as_ref, ad_ref, hb_ref, azs_ref, azd_ref):
    h = jnp.dot(x_ref[...], w_ref[...], preferred_element_type=jnp.float32)
    asrc = jnp.dot(h, as_ref[...], preferred_element_type=jnp.float32)
    adst = jnp.dot(h, ad_ref[...], preferred_element_type=jnp.float32)
    hb_ref[...] = h.astype(jnp.bfloat16)
    azs_ref[...] = asrc
    azd_ref[...] = adst


def _tc_mid_body(acc_ref, hb_ref, azs_ref, azd_ref, b1_ref, w2_ref,
                 a2s_ref, a2d_ref, r_ref, hb2_ref, az2s_ref, az2d_ref):
    h = hb_ref[...].astype(jnp.float32)
    asrc = azs_ref[...]
    adst = azd_ref[...]
    num = acc_ref[0][:, :64] + acc_ref[1][:, :64]
    den8 = acc_ref[0][:, 64:72] + acc_ref[1][:, 64:72]
    exs = _leaky_exp(asrc + adst)
    rmat = r_ref[...]
    num = num + jnp.dot(exs, rmat, preferred_element_type=jnp.float32) * h
    den = jnp.dot(den8 + exs, rmat, preferred_element_type=jnp.float32)
    v = num / (den + 1e-16) + b1_ref[...]
    h1 = jnp.where(v > 0, v, jnp.exp(jnp.minimum(v, 0.0)) - 1.0)
    h2 = jnp.dot(h1, w2_ref[...], preferred_element_type=jnp.float32)
    as2 = jnp.dot(h2, a2s_ref[...], preferred_element_type=jnp.float32)
    ad2 = jnp.dot(h2, a2d_ref[...], preferred_element_type=jnp.float32)
    hb2_ref[...] = h2.astype(jnp.bfloat16)
    pad7 = jnp.zeros((h2.shape[0], 7), jnp.float32)
    az2s_ref[...] = jnp.concatenate([as2, pad7], axis=1)
    az2d_ref[...] = jnp.concatenate([ad2, pad7], axis=1)


def _tc_post_body(acc_ref, hb2_ref, az2s_ref, az2d_ref, b2_ref, out_ref):
    h2 = hb2_ref[...].astype(jnp.float32)
    as2 = az2s_ref[...][:, 0:1]
    ad2 = az2d_ref[...][:, 0:1]
    num = acc_ref[0][:, :64] + acc_ref[1][:, :64]
    den = acc_ref[0][:, 64:65] + acc_ref[1][:, 64:65]
    exs = _leaky_exp(as2 + ad2)
    logits = (num + exs * h2) / (den + exs + 1e-16) + b2_ref[...]
    m = jnp.max(logits, axis=1, keepdims=True)
    out_ref[...] = logits - m - jnp.log(
        jnp.sum(jnp.exp(logits - m), axis=1, keepdims=True))


def _zero_mbuf(mbuf):
    iota = lax.iota(jnp.int32, 16)
    zeros = jnp.zeros((16,), jnp.float32)

    def body(i, carry):
        mbuf[i, pl.ds(0, 16)] = zeros
        mbuf[i, pl.ds(16, 16)] = zeros
        mbuf[i, pl.ds(32, 16)] = zeros
        mbuf[i, pl.ds(48, 16)] = zeros
        plsc.store_scatter(mbuf, [jnp.full((16,), i, jnp.int32), 56 + iota],
                           zeros)
        return carry

    lax.fori_loop(0, CK, body, 0)


def _zero_spmem(mbuf, acc_sp, s):
    base = s * RPT
    off = 0
    while off + CK <= RPT:
        pltpu.sync_copy(mbuf, acc_sp.at[pl.ds(base + off, CK), :])
        off += CK
    if off < RPT:
        pltpu.sync_copy(mbuf.at[pl.ds(0, RPT - off), :],
                        acc_sp.at[pl.ds(base + off, RPT - off), :])


def _copy_out(acc_sp, acc_hbm, c, s):
    base = s * RPT
    pltpu.sync_copy(acc_sp.at[pl.ds(base, RPT), :],
                    acc_hbm.at[c, pl.ds(base, RPT), :])


def _vbcast(v, idx16):
    dn = lax.GatherDimensionNumbers(
        offset_dims=(), collapsed_slice_dims=(0,), start_index_map=(0,))
    return lax.gather(v, idx16[:, None], dn, (1,),
                      mode=lax.GatherScatterMode.PROMISE_IN_BOUNDS)


def _zero_idx_tail(srci, dsti):
    zi = jnp.zeros((16,), jnp.int32)
    for r in (CPT, CPT + 1):
        for k in range(CK // 16):
            srci[r, pl.ds(16 * k, 16)] = zi
            dsti[r, pl.ds(16 * k, 16)] = zi


def _unpack_mul_store(hbuf, mbuf, erow, rsplat, exv, pidx, e2):
    for k in range(2):
        hv32 = hbuf[erow, pl.ds(32 * k, 32)]
        pa, pb = plsc.unpack(hv32, format=plsc.PackFormat.INTERLEAVED)
        exb = _vbcast(exv, pidx[k])
        plsc.store_scatter(mbuf, [rsplat, e2 + 32 * k], pa * exb)
        plsc.store_scatter(mbuf, [rsplat, e2 + 32 * k + 1], pb * exb)


def _compute_chunk1(hbuf, arows, brows, mbuf, iota, step8, col8, bidx):
    c64p = 64 + col8
    e2 = 2 * iota

    def body2(p, carry2):
        r16 = 2 * p + step8
        a = plsc.load_gather(arows, [r16, col8])
        b = plsc.load_gather(brows, [r16, col8])
        ex = _leaky_exp(a + b)
        plsc.store_scatter(mbuf, [r16, c64p], ex)
        for edge in range(2):
            erow = 2 * p + edge
            rsplat = jnp.zeros((16,), jnp.int32) + erow
            _unpack_mul_store(hbuf, mbuf, erow, rsplat, ex, bidx[edge], e2)
        return carry2

    lax.fori_loop(0, CK // 2, body2, 0)


def _compute_chunk2(hbuf, arows, brows, mbuf, iota, c64, c0, eidx):
    e2 = 2 * iota

    def group_body(g, carry2):
        rows16 = 16 * g + iota
        a = plsc.load_gather(arows, [rows16, c0])
        b = plsc.load_gather(brows, [rows16, c0])
        ex = _leaky_exp(a + b)
        plsc.store_scatter(mbuf, [rows16, c64], ex)
        for e in range(16):
            erow = 16 * g + e
            rsplat = jnp.zeros((16,), jnp.int32) + erow
            _unpack_mul_store(hbuf, mbuf, erow, rsplat, ex,
                              [eidx[e], eidx[e]], e2)
        return carry2

    lax.fori_loop(0, CK // 16, group_body, 0)


def _edge_pass_pipelined(compute, srcc, dstc, hb_hbm, azs_hbm, azd_hbm, acc_hbm,
                         srci, dsti, hbufA, hbufB, arowsA, arowsB,
                         browsA, browsB, mbuf, acc_sp, gsemA, gsemB):
    c = lax.axis_index("c")
    s = lax.axis_index("s")
    t = c * NS + s

    _zero_mbuf(mbuf)
    _zero_spmem(mbuf, acc_sp, s)
    _zero_idx_tail(srci, dsti)
    plsc.subcore_barrier()

    pltpu.sync_copy(srcc.at[t], srci.at[pl.ds(0, CPT), :])
    pltpu.sync_copy(dstc.at[t], dsti.at[pl.ds(0, CPT), :])

    def fire_gather(j, hbuf, arows, brows, gsem):
        pltpu.async_copy(hb_hbm.at[srci.at[j]], hbuf, gsem)
        pltpu.async_copy(azs_hbm.at[srci.at[j]], arows, gsem)
        pltpu.async_copy(azd_hbm.at[dsti.at[j]], brows, gsem)

    def wait_gather(j, hbuf, arows, brows, gsem):
        pltpu.make_async_copy(hb_hbm.at[srci.at[j]], hbuf, gsem).wait()
        pltpu.make_async_copy(azs_hbm.at[srci.at[j]], arows, gsem).wait()
        pltpu.make_async_copy(azd_hbm.at[dsti.at[j]], brows, gsem).wait()

    fire_gather(0, hbufA, arowsA, browsA, gsemA)

    def body(jj, carry):
        j0 = 2 * jj
        j1 = 2 * jj + 1
        wait_gather(j0, hbufA, arowsA, browsA, gsemA)
        fire_gather(j1, hbufB, arowsB, browsB, gsemB)
        compute(hbufA, arowsA, browsA, mbuf)
        pltpu.sync_copy(mbuf, acc_sp.at[dsti.at[j0]], add=True)
        wait_gather(j1, hbufB, arowsB, browsB, gsemB)
        fire_gather(j0 + 2, hbufA, arowsA, browsA, gsemA)
        compute(hbufB, arowsB, browsB, mbuf)
        pltpu.sync_copy(mbuf, acc_sp.at[dsti.at[j1]], add=True)
        return carry

    lax.fori_loop(0, CPT // 2, body, 0)

    wait_gather(CPT, hbufA, arowsA, browsA, gsemA)
    plsc.subcore_barrier()
    _copy_out(acc_sp, acc_hbm, c, s)


def _edge_pass1(srcc, dstc, hb_hbm, azs_hbm, azd_hbm, acc_hbm,
                srci, dsti, hbufA, hbufB, arowsA, arowsB,
                browsA, browsB, mbuf, acc_sp, gsemA, gsemB):
    iota = lax.iota(jnp.int32, 16)
    step8 = jnp.where(iota >= 8, 1, 0)
    col8 = iota & 7
    q4 = iota // 4
    bidx = [[8 * edge + 4 * k + q4 for k in range(2)] for edge in range(2)]

    def comp(hbuf, arows, brows, mb):
        _compute_chunk1(hbuf, arows, brows, mb, iota, step8, col8, bidx)

    _edge_pass_pipelined(comp, srcc, dstc, hb_hbm, azs_hbm, azd_hbm, acc_hbm,
                         srci, dsti, hbufA, hbufB, arowsA, arowsB,
                         browsA, browsB, mbuf, acc_sp, gsemA, gsemB)


def _edge_pass2(srcc, dstc, hb_hbm, azs_hbm, azd_hbm, acc_hbm,
                srci, dsti, hbufA, hbufB, arowsA, arowsB,
                browsA, browsB, mbuf, acc_sp, gsemA, gsemB):
    iota = lax.iota(jnp.int32, 16)
    c64 = jnp.full((16,), 64, jnp.int32)
    c0 = jnp.zeros((16,), jnp.int32)
    eidx = [jnp.full((16,), e, jnp.int32) for e in range(16)]

    def comp(hbuf, arows, brows, mb):
        _compute_chunk2(hbuf, arows, brows, mb, iota, c64, c0, eidx)

    _edge_pass_pipelined(comp, srcc, dstc, hb_hbm, azs_hbm, azd_hbm, acc_hbm,
                         srci, dsti, hbufA, hbufB, arowsA, arowsB,
                         browsA, browsB, mbuf, acc_sp, gsemA, gsemB)


@functools.lru_cache(maxsize=None)
def _sc_kernels():
    mesh = plsc.VectorSubcoreMesh(
        core_axis_name="c", subcore_axis_name="s",
        num_cores=NC, num_subcores=NS)
    scratch = [
        pltpu.VMEM((CPT + 2, CK), jnp.int32),
        pltpu.VMEM((CPT + 2, CK), jnp.int32),
        pltpu.VMEM((CK, 64), jnp.bfloat16),
        pltpu.VMEM((CK, 64), jnp.bfloat16),
        pltpu.VMEM((CK, 8), jnp.float32),
        pltpu.VMEM((CK, 8), jnp.float32),
        pltpu.VMEM((CK, 8), jnp.float32),
        pltpu.VMEM((CK, 8), jnp.float32),
        pltpu.VMEM((CK, ACCW), jnp.float32),
        pltpu.VMEM_SHARED((NPAD, ACCW), jnp.float32),
        pltpu.SemaphoreType.DMA,
        pltpu.SemaphoreType.DMA,
    ]
    mk = functools.partial(
        pl.kernel,
        out_type=jax.ShapeDtypeStruct((NC, NPAD, ACCW), jnp.float32),
        mesh=mesh, scratch_types=scratch,
        compiler_params=pltpu.CompilerParams(
            needs_layout_passes=False, use_tc_tiling_on_sc=False))
    return mk(_edge_pass1), mk(_edge_pass2)


def _node_specs(widths):
    return [pl.BlockSpec((BN, w), lambda i: (i, 0)) for w in widths]


def kernel(x, edge_index, W1, att_src1, att_dst1, b1, W2, att_src2,
           att_dst2, b2):
    f32 = jnp.float32
    eye8 = jnp.eye(8, dtype=f32)
    a_s = (eye8[:, None, :] * att_src1[:, :, None]).reshape(64, 8)
    a_d = (eye8[:, None, :] * att_dst1[:, :, None]).reshape(64, 8)
    rmat = jnp.repeat(eye8, 8, axis=1)

    hb, azs, azd = pl.pallas_call(
        _tc_pre_body,
        grid=(GRID,),
        in_specs=[
            pl.BlockSpec((BN, D_IN), lambda i: (i, 0)),
            pl.BlockSpec((D_IN, 64), lambda i: (0, 0)),
            pl.BlockSpec((64, 8), lambda i: (0, 0)),
            pl.BlockSpec((64, 8), lambda i: (0, 0)),
        ],
        out_specs=_node_specs([64, 8, 8]),
        out_shape=[
            jax.ShapeDtypeStruct((N, 64), jnp.bfloat16),
            jax.ShapeDtypeStruct((N, 8), f32),
            jax.ShapeDtypeStruct((N, 8), f32),
        ],
    )(x, W1, a_s, a_d)

    src = edge_index[0]
    dst = edge_index[1]
    pad = EPAD - E
    srcc = jnp.concatenate([src, jnp.zeros((pad,), jnp.int32)])
    srcc = srcc.reshape(NT, CPT, CK)
    dstc = jnp.concatenate([dst, jnp.full((pad,), N, jnp.int32)])
    dstc = dstc.reshape(NT, CPT, CK)
    azd_p = jnp.pad(azd, ((0, 16), (0, 0)))

    edge_pass1, edge_pass2 = _sc_kernels()
    acc1 = edge_pass1(srcc, dstc, hb, azs, azd_p)

    hb2, az2s, az2d = pl.pallas_call(
        _tc_mid_body,
        grid=(GRID,),
        in_specs=[
            pl.BlockSpec((NC, BN, ACCW), lambda i: (0, i, 0)),
            pl.BlockSpec((BN, 64), lambda i: (i, 0)),
            pl.BlockSpec((BN, 8), lambda i: (i, 0)),
            pl.BlockSpec((BN, 8), lambda i: (i, 0)),
            pl.BlockSpec((1, 64), lambda i: (0, 0)),
            pl.BlockSpec((64, 64), lambda i: (0, 0)),
            pl.BlockSpec((64, 1), lambda i: (0, 0)),
            pl.BlockSpec((64, 1), lambda i: (0, 0)),
            pl.BlockSpec((8, 64), lambda i: (0, 0)),
        ],
        out_specs=_node_specs([64, 8, 8]),
        out_shape=[
            jax.ShapeDtypeStruct((N, 64), jnp.bfloat16),
            jax.ShapeDtypeStruct((N, 8), f32),
            jax.ShapeDtypeStruct((N, 8), f32),
        ],
    )(acc1[:, :N, :], hb, azs, azd, b1.reshape(1, 64), W2,
      att_src2.reshape(64, 1), att_dst2.reshape(64, 1), rmat)

    az2d_p = jnp.pad(az2d, ((0, 16), (0, 0)))
    acc2 = edge_pass2(srcc, dstc, hb2, az2s, az2d_p)

    out = pl.pallas_call(
        _tc_post_body,
        grid=(GRID,),
        in_specs=[
            pl.BlockSpec((NC, BN, ACCW), lambda i: (0, i, 0)),
            pl.BlockSpec((BN, 64), lambda i: (i, 0)),
            pl.BlockSpec((BN, 8), lambda i: (i, 0)),
            pl.BlockSpec((BN, 8), lambda i: (i, 0)),
            pl.BlockSpec((1, 64), lambda i: (0, 0)),
        ],
        out_specs=pl.BlockSpec((BN, D_OUT), lambda i: (i, 0)),
        out_shape=jax.ShapeDtypeStruct((N, D_OUT), f32),
    )(acc2[:, :N, :], hb2, az2s, az2d, b2.reshape(1, 64))

    return out


# speedup vs baseline: 1.0115x; 1.0115x over previous
import functools

import jax
import jax.numpy as jnp
from jax import lax
from jax.experimental import pallas as pl
from jax.experimental.pallas import tpu as pltpu
from jax.experimental.pallas import tpu_sc as plsc

N = 10000
E = 320000
D_IN = 128
HID = 8
HEADS = 8
D_OUT = 64

NC = 2
NS = 16
NT = NC * NS
CK = 128
CPT = 80
EPAD = NT * CPT * CK
NPAD = 10112
RPT = NPAD // NS
ACCW = 72
BN = 1000
GRID = N // BN


def _leaky_exp(z):
    return jnp.exp(jnp.maximum(z, 0.2 * z))


def _tc_pre_body(x_ref, w_ref, as_ref, ad_ref, hb_ref, az_ref):
    h = jnp.dot(x_ref[...], w_ref[...], preferred_element_type=jnp.float32)
    asrc = jnp.dot(h, as_ref[...], preferred_element_type=jnp.float32)
    adst = jnp.dot(h, ad_ref[...], preferred_element_type=jnp.float32)
    hb_ref[...] = h.astype(jnp.bfloat16)
    az_ref[...] = jnp.concatenate([asrc, adst], axis=1)


def _tc_mid_body(acc_ref, hb_ref, az_ref, b1_ref, w2_ref, a2s_ref, a2d_ref,
                 r_ref, hb2_ref, az2_ref):
    h = hb_ref[...].astype(jnp.float32)
    asrc = az_ref[...][:, :8]
    adst = az_ref[...][:, 8:16]
    num = acc_ref[0][:, :64] + acc_ref[1][:, :64]
    den8 = acc_ref[0][:, 64:72] + acc_ref[1][:, 64:72]
    exs = _leaky_exp(asrc + adst)
    rmat = r_ref[...]
    num = num + jnp.dot(exs, rmat, preferred_element_type=jnp.float32) * h
    den = jnp.dot(den8 + exs, rmat, preferred_element_type=jnp.float32)
    v = num / (den + 1e-16) + b1_ref[...]
    h1 = jnp.where(v > 0, v, jnp.exp(jnp.minimum(v, 0.0)) - 1.0)
    h2 = jnp.dot(h1, w2_ref[...], preferred_element_type=jnp.float32)
    as2 = jnp.dot(h2, a2s_ref[...], preferred_element_type=jnp.float32)
    ad2 = jnp.dot(h2, a2d_ref[...], preferred_element_type=jnp.float32)
    hb2_ref[...] = h2.astype(jnp.bfloat16)
    pad7 = jnp.zeros((h2.shape[0], 7), jnp.float32)
    az2_ref[...] = jnp.concatenate([as2, pad7, ad2, pad7], axis=1)


def _tc_post_body(acc_ref, hb2_ref, az2_ref, b2_ref, out_ref):
    h2 = hb2_ref[...].astype(jnp.float32)
    as2 = az2_ref[...][:, 0:1]
    ad2 = az2_ref[...][:, 8:9]
    num = acc_ref[0][:, :64] + acc_ref[1][:, :64]
    den = acc_ref[0][:, 64:65] + acc_ref[1][:, 64:65]
    exs = _leaky_exp(as2 + ad2)
    logits = (num + exs * h2) / (den + exs + 1e-16) + b2_ref[...]
    m = jnp.max(logits, axis=1, keepdims=True)
    out_ref[...] = logits - m - jnp.log(
        jnp.sum(jnp.exp(logits - m), axis=1, keepdims=True))


def _zero_mbuf(mbuf):
    iota = lax.iota(jnp.int32, 16)
    zeros = jnp.zeros((16,), jnp.float32)

    def body(i, carry):
        mbuf[i, pl.ds(0, 16)] = zeros
        mbuf[i, pl.ds(16, 16)] = zeros
        mbuf[i, pl.ds(32, 16)] = zeros
        mbuf[i, pl.ds(48, 16)] = zeros
        plsc.store_scatter(mbuf, [jnp.full((16,), i, jnp.int32), 56 + iota],
                           zeros)
        return carry

    lax.fori_loop(0, CK, body, 0)


def _zero_spmem(mbuf, acc_sp, s):
    base = s * RPT
    off = 0
    while off + CK <= RPT:
        pltpu.sync_copy(mbuf, acc_sp.at[pl.ds(base + off, CK), :])
        off += CK
    if off < RPT:
        pltpu.sync_copy(mbuf.at[pl.ds(0, RPT - off), :],
                        acc_sp.at[pl.ds(base + off, RPT - off), :])


def _copy_out(acc_sp, acc_hbm, c, s):
    base = s * RPT
    pltpu.sync_copy(acc_sp.at[pl.ds(base, RPT), :],
                    acc_hbm.at[c, pl.ds(base, RPT), :])


def _vbcast(v, idx16):
    dn = lax.GatherDimensionNumbers(
        offset_dims=(), collapsed_slice_dims=(0,), start_index_map=(0,))
    return lax.gather(v, idx16[:, None], dn, (1,),
                      mode=lax.GatherScatterMode.PROMISE_IN_BOUNDS)


def _zero_idx_tail(srci, dsti):
    zi = jnp.zeros((16,), jnp.int32)
    for r in (CPT, CPT + 1):
        for k in range(CK // 16):
            srci[r, pl.ds(16 * k, 16)] = zi
            dsti[r, pl.ds(16 * k, 16)] = zi


def _unpack_mul_store(hbuf, mbuf, erow, rsplat, exv, pidx, e2):
    for k in range(2):
        hv32 = hbuf[erow, pl.ds(32 * k, 32)]
        pa, pb = plsc.unpack(hv32, format=plsc.PackFormat.INTERLEAVED)
        exb = _vbcast(exv, pidx[k])
        plsc.store_scatter(mbuf, [rsplat, e2 + 32 * k], pa * exb)
        plsc.store_scatter(mbuf, [rsplat, e2 + 32 * k + 1], pb * exb)


def _compute_chunk1(hbuf, arows, brows, mbuf, iota, step8, col8, bidx):
    c64p = 64 + col8
    e2 = 2 * iota

    def body2(p, carry2):
        r16 = 2 * p + step8
        a = plsc.load_gather(arows, [r16, col8])
        b = plsc.load_gather(brows, [r16, 8 + col8])
        ex = _leaky_exp(a + b)
        plsc.store_scatter(mbuf, [r16, c64p], ex)
        for edge in range(2):
            erow = 2 * p + edge
            rsplat = jnp.zeros((16,), jnp.int32) + erow
            _unpack_mul_store(hbuf, mbuf, erow, rsplat, ex, bidx[edge], e2)
        return carry2

    lax.fori_loop(0, CK // 2, body2, 0)


def _compute_chunk2(hbuf, arows, brows, mbuf, iota, c64, c0, c8, eidx):
    e2 = 2 * iota

    def group_body(g, carry2):
        rows16 = 16 * g + iota
        a = plsc.load_gather(arows, [rows16, c0])
        b = plsc.load_gather(brows, [rows16, c8])
        ex = _leaky_exp(a + b)
        plsc.store_scatter(mbuf, [rows16, c64], ex)
        for e in range(16):
            erow = 16 * g + e
            rsplat = jnp.zeros((16,), jnp.int32) + erow
            _unpack_mul_store(hbuf, mbuf, erow, rsplat, ex,
                              [eidx[e], eidx[e]], e2)
        return carry2

    lax.fori_loop(0, CK // 16, group_body, 0)


def _edge_pass_pipelined(compute, srcc, dstc, hb_hbm, az_hbm, acc_hbm,
                         srci, dsti, hbufA, hbufB, arowsA, arowsB,
                         browsA, browsB, mbuf, acc_sp, gsemA, gsemB):
    c = lax.axis_index("c")
    s = lax.axis_index("s")
    t = c * NS + s

    _zero_mbuf(mbuf)
    _zero_spmem(mbuf, acc_sp, s)
    _zero_idx_tail(srci, dsti)
    plsc.subcore_barrier()

    pltpu.sync_copy(srcc.at[t], srci.at[pl.ds(0, CPT), :])
    pltpu.sync_copy(dstc.at[t], dsti.at[pl.ds(0, CPT), :])

    def fire_gather(j, hbuf, arows, brows, gsem):
        pltpu.async_copy(hb_hbm.at[srci.at[j]], hbuf, gsem)
        pltpu.async_copy(az_hbm.at[srci.at[j]], arows, gsem)
        pltpu.async_copy(az_hbm.at[dsti.at[j]], brows, gsem)

    def wait_gather(j, hbuf, arows, brows, gsem):
        pltpu.make_async_copy(hb_hbm.at[srci.at[j]], hbuf, gsem).wait()
        pltpu.make_async_copy(az_hbm.at[srci.at[j]], arows, gsem).wait()
        pltpu.make_async_copy(az_hbm.at[dsti.at[j]], brows, gsem).wait()

    fire_gather(0, hbufA, arowsA, browsA, gsemA)

    def body(jj, carry):
        j0 = 2 * jj
        j1 = 2 * jj + 1
        wait_gather(j0, hbufA, arowsA, browsA, gsemA)
        fire_gather(j1, hbufB, arowsB, browsB, gsemB)
        compute(hbufA, arowsA, browsA, mbuf)
        pltpu.sync_copy(mbuf, acc_sp.at[dsti.at[j0]], add=True)
        wait_gather(j1, hbufB, arowsB, browsB, gsemB)
        fire_gather(j0 + 2, hbufA, arowsA, browsA, gsemA)
        compute(hbufB, arowsB, browsB, mbuf)
        pltpu.sync_copy(mbuf, acc_sp.at[dsti.at[j1]], add=True)
        return carry

    lax.fori_loop(0, CPT // 2, body, 0)

    wait_gather(CPT, hbufA, arowsA, browsA, gsemA)
    plsc.subcore_barrier()
    _copy_out(acc_sp, acc_hbm, c, s)


def _edge_pass1(srcc, dstc, hb_hbm, az_hbm, acc_hbm,
                srci, dsti, hbufA, hbufB, arowsA, arowsB,
                browsA, browsB, mbuf, acc_sp, gsemA, gsemB):
    iota = lax.iota(jnp.int32, 16)
    step8 = jnp.where(iota >= 8, 1, 0)
    col8 = iota & 7
    q4 = iota // 4
    bidx = [[8 * edge + 4 * k + q4 for k in range(2)] for edge in range(2)]

    def comp(hbuf, arows, brows, mb):
        _compute_chunk1(hbuf, arows, brows, mb, iota, step8, col8, bidx)

    _edge_pass_pipelined(comp, srcc, dstc, hb_hbm, az_hbm, acc_hbm,
                         srci, dsti, hbufA, hbufB, arowsA, arowsB,
                         browsA, browsB, mbuf, acc_sp, gsemA, gsemB)


def _edge_pass2(srcc, dstc, hb_hbm, az_hbm, acc_hbm,
                srci, dsti, hbufA, hbufB, arowsA, arowsB,
                browsA, browsB, mbuf, acc_sp, gsemA, gsemB):
    iota = lax.iota(jnp.int32, 16)
    c64 = jnp.full((16,), 64, jnp.int32)
    c0 = jnp.zeros((16,), jnp.int32)
    c8 = jnp.full((16,), 8, jnp.int32)
    eidx = [jnp.full((16,), e, jnp.int32) for e in range(16)]

    def comp(hbuf, arows, brows, mb):
        _compute_chunk2(hbuf, arows, brows, mb, iota, c64, c0, c8, eidx)

    _edge_pass_pipelined(comp, srcc, dstc, hb_hbm, az_hbm, acc_hbm,
                         srci, dsti, hbufA, hbufB, arowsA, arowsB,
                         browsA, browsB, mbuf, acc_sp, gsemA, gsemB)


@functools.lru_cache(maxsize=None)
def _sc_kernels():
    mesh = plsc.VectorSubcoreMesh(
        core_axis_name="c", subcore_axis_name="s",
        num_cores=NC, num_subcores=NS)
    scratch = [
        pltpu.VMEM((CPT + 2, CK), jnp.int32),
        pltpu.VMEM((CPT + 2, CK), jnp.int32),
        pltpu.VMEM((CK, 64), jnp.bfloat16),
        pltpu.VMEM((CK, 64), jnp.bfloat16),
        pltpu.VMEM((CK, 16), jnp.float32),
        pltpu.VMEM((CK, 16), jnp.float32),
        pltpu.VMEM((CK, 16), jnp.float32),
        pltpu.VMEM((CK, 16), jnp.float32),
        pltpu.VMEM((CK, ACCW), jnp.float32),
        pltpu.VMEM_SHARED((NPAD, ACCW), jnp.float32),
        pltpu.SemaphoreType.DMA,
        pltpu.SemaphoreType.DMA,
    ]
    mk = functools.partial(
        pl.kernel,
        out_type=jax.ShapeDtypeStruct((NC, NPAD, ACCW), jnp.float32),
        mesh=mesh, scratch_types=scratch,
        compiler_params=pltpu.CompilerParams(
            needs_layout_passes=False, use_tc_tiling_on_sc=False))
    return mk(_edge_pass1), mk(_edge_pass2)


def _node_specs(widths):
    return [pl.BlockSpec((BN, w), lambda i: (i, 0)) for w in widths]


def kernel(x, edge_index, W1, att_src1, att_dst1, b1, W2, att_src2,
           att_dst2, b2):
    f32 = jnp.float32
    eye8 = jnp.eye(8, dtype=f32)
    a_s = (eye8[:, None, :] * att_src1[:, :, None]).reshape(64, 8)
    a_d = (eye8[:, None, :] * att_dst1[:, :, None]).reshape(64, 8)
    rmat = jnp.repeat(eye8, 8, axis=1)

    hb, az = pl.pallas_call(
        _tc_pre_body,
        grid=(GRID,),
        in_specs=[
            pl.BlockSpec((BN, D_IN), lambda i: (i, 0)),
            pl.BlockSpec((D_IN, 64), lambda i: (0, 0)),
            pl.BlockSpec((64, 8), lambda i: (0, 0)),
            pl.BlockSpec((64, 8), lambda i: (0, 0)),
        ],
        out_specs=_node_specs([64, 16]),
        out_shape=[
            jax.ShapeDtypeStruct((N, 64), jnp.bfloat16),
            jax.ShapeDtypeStruct((N, 16), f32),
        ],
    )(x, W1, a_s, a_d)

    src = edge_index[0]
    dst = edge_index[1]
    pad = EPAD - E
    srcc = jnp.concatenate([src, jnp.zeros((pad,), jnp.int32)])
    srcc = srcc.reshape(NT, CPT, CK)
    dstc = jnp.concatenate([dst, jnp.full((pad,), N, jnp.int32)])
    dstc = dstc.reshape(NT, CPT, CK)
    az_p = jnp.pad(az, ((0, 16), (0, 0)))
    hb_p = jnp.pad(hb, ((0, 16), (0, 0)))

    edge_pass1, edge_pass2 = _sc_kernels()
    acc1 = edge_pass1(srcc, dstc, hb_p, az_p)

    hb2, az2 = pl.pallas_call(
        _tc_mid_body,
        grid=(GRID,),
        in_specs=[
            pl.BlockSpec((NC, BN, ACCW), lambda i: (0, i, 0)),
            pl.BlockSpec((BN, 64), lambda i: (i, 0)),
            pl.BlockSpec((BN, 16), lambda i: (i, 0)),
            pl.BlockSpec((1, 64), lambda i: (0, 0)),
            pl.BlockSpec((64, 64), lambda i: (0, 0)),
            pl.BlockSpec((64, 1), lambda i: (0, 0)),
            pl.BlockSpec((64, 1), lambda i: (0, 0)),
            pl.BlockSpec((8, 64), lambda i: (0, 0)),
        ],
        out_specs=_node_specs([64, 16]),
        out_shape=[
            jax.ShapeDtypeStruct((N, 64), jnp.bfloat16),
            jax.ShapeDtypeStruct((N, 16), f32),
        ],
    )(acc1[:, :N, :], hb, az, b1.reshape(1, 64), W2,
      att_src2.reshape(64, 1), att_dst2.reshape(64, 1), rmat)

    az2_p = jnp.pad(az2, ((0, 16), (0, 0)))
    hb2_p = jnp.pad(hb2, ((0, 16), (0, 0)))
    acc2 = edge_pass2(srcc, dstc, hb2_p, az2_p)

    out = pl.pallas_call(
        _tc_post_body,
        grid=(GRID,),
        in_specs=[
            pl.BlockSpec((NC, BN, ACCW), lambda i: (0, i, 0)),
            pl.BlockSpec((BN, 64), lambda i: (i, 0)),
            pl.BlockSpec((BN, 16), lambda i: (i, 0)),
            pl.BlockSpec((1, 64), lambda i: (0, 0)),
        ],
        out_specs=pl.BlockSpec((BN, D_OUT), lambda i: (i, 0)),
        out_shape=jax.ShapeDtypeStruct((N, D_OUT), f32),
    )(acc2[:, :N, :], hb2, az2, b2.reshape(1, 64))

    return out
